# Initial kernel scaffold; baseline (speedup 1.0000x reference)
import jax
import jax.numpy as jnp
from jax.experimental import pallas as pl


def kernel(x, word2vec, s_f, edge_index, W_e, b_e, W_el, b_el, W_a, b_a, W_al, b_al, W_n, b_n, W_nl, b_nl):
    raise NotImplementedError("write your pallas kernel here")


# speedup vs baseline: 3.8082x; 3.8082x over previous
import functools

import jax
import jax.numpy as jnp
from jax import lax
from jax.experimental import pallas as pl
from jax.experimental.pallas import tpu as pltpu
from jax.experimental.pallas import tpu_sc as plsc

F32 = jnp.float32
NC = 2
NS = 16
NW = NC * NS
CK = 80


def _mesh():
    return plsc.VectorSubcoreMesh(
        core_axis_name="c", subcore_axis_name="s", num_cores=NC, num_subcores=NS)


_SC_PARAMS = pltpu.CompilerParams(
    needs_layout_passes=False, use_tc_tiling_on_sc=False)


def _t1_tables(x, w2v, WeT, WeB, WelT, WelB, belp):
    N, D = x.shape
    DWP = WelT.shape[1]
    SRCW = 2 * D + DWP
    DSTW = D + DWP
    NB = 5
    RB = N // NB

    def body(x_ref, wv_ref, wet, web, welt, welb, bel, src_ref, dst_ref):
        xb = x_ref[...]
        wv = wv_ref[...]
        src_ref[:, 0:D] = jnp.dot(xb, wet[...], preferred_element_type=F32)
        src_ref[:, D:2 * D] = xb
        src_ref[:, 2 * D:SRCW] = (
            jnp.dot(wv, welt[...], preferred_element_type=F32) + bel[...])
        dst_ref[:, 0:D] = jnp.dot(xb, web[...], preferred_element_type=F32)
        dst_ref[:, D:DSTW] = jnp.dot(wv, welb[...], preferred_element_type=F32)

    return pl.pallas_call(
        body,
        grid=(NB,),
        in_specs=[
            pl.BlockSpec((RB, x.shape[1]), lambda i: (i, 0)),
            pl.BlockSpec((RB, w2v.shape[1]), lambda i: (i, 0)),
            pl.BlockSpec(WeT.shape, lambda i: (0, 0)),
            pl.BlockSpec(WeB.shape, lambda i: (0, 0)),
            pl.BlockSpec(WelT.shape, lambda i: (0, 0)),
            pl.BlockSpec(WelB.shape, lambda i: (0, 0)),
            pl.BlockSpec(belp.shape, lambda i: (0, 0)),
        ],
        out_specs=[
            pl.BlockSpec((RB, SRCW), lambda i: (i, 0)),
            pl.BlockSpec((RB, DSTW), lambda i: (i, 0)),
        ],
        out_shape=[
            jax.ShapeDtypeStruct((N, SRCW), F32),
            jax.ShapeDtypeStruct((N, DSTW), F32),
        ],
    )(x, w2v, WeT, WeB, WelT, WelB, belp)


def _t2_cedge(s_f, WeM, be2):
    E, DS = s_f.shape
    D = WeM.shape[1]
    EB = 8000
    NB = E // EB

    def body(sf_ref, w_ref, b_ref, c_ref):
        c_ref[...] = (
            jnp.dot(sf_ref[...], w_ref[...], preferred_element_type=F32)
            + b_ref[...])

    return pl.pallas_call(
        body,
        grid=(NB,),
        in_specs=[
            pl.BlockSpec((EB, DS), lambda i: (i, 0)),
            pl.BlockSpec(WeM.shape, lambda i: (0, 0)),
            pl.BlockSpec(be2.shape, lambda i: (0, 0)),
        ],
        out_specs=pl.BlockSpec((EB, D), lambda i: (i, 0)),
        out_shape=jax.ShapeDtypeStruct((E, D), F32),
    )(s_f, WeM, be2)


def _t3_recip(s16):
    _, N, L = s16.shape

    def body(s_ref, o_ref):
        s = s_ref[0] + s_ref[1]
        lane = lax.broadcasted_iota(jnp.int32, (N, L), 1)
        tot = jnp.sum(jnp.where(lane == 0, s, 0.0), axis=1)
        totl = jnp.sum(jnp.where(lane == 1, s, 0.0), axis=1)
        r0 = (1.0 / (tot + 1e-9)).reshape(1, N)
        r1 = (1.0 / (totl + 1e-9)).reshape(1, N)
        o_ref[...] = jnp.concatenate([r0, r1], axis=0)

    return pl.pallas_call(
        body, out_shape=jax.ShapeDtypeStruct((2, N), F32))(s16)


def _t4_node(x, w2v, zf, za, zb, zc,
             WnT, WnB, bn2, WnlT, WnlBA, WnlBB, WnlBC, bnl2):
    N, D = x.shape
    DW = w2v.shape[1]
    NB = 5
    RB = N // NB

    def body(x_ref, wv_ref, z0r, z1r, za0r, za1r, zb0r, zb1r, zc0r, zc1r,
             wnt, wnb, bn, wnlt, wnlba, wnlbb, wnlbc, bnl, o1_ref, o2_ref):
        z = z0r[...] + z1r[...]
        o1_ref[...] = jnp.maximum(
            jnp.dot(x_ref[...], wnt[...], preferred_element_type=F32)
            + jnp.dot(z, wnb[...], preferred_element_type=F32) + bn[...], 0.0)
        zav = za0r[...] + za1r[...]
        zbv = zb0r[...] + zb1r[...]
        zcv = zc0r[...] + zc1r[...]
        o2_ref[...] = jnp.maximum(
            jnp.dot(wv_ref[...], wnlt[...], preferred_element_type=F32)
            + jnp.dot(zav, wnlba[...], preferred_element_type=F32)
            + jnp.dot(zbv, wnlbb[...], preferred_element_type=F32)
            + jnp.dot(zcv, wnlbc[...], preferred_element_type=F32)
            + bnl[...], 0.0)

    row = lambda a: pl.BlockSpec((RB, a.shape[-1]), lambda i: (i, 0))
    full = lambda a: pl.BlockSpec(a.shape, lambda i: tuple(0 for _ in a.shape))
    return pl.pallas_call(
        body,
        grid=(NB,),
        in_specs=[
            row(x), row(w2v),
            row(zf[0]), row(zf[1]), row(za[0]), row(za[1]),
            row(zb[0]), row(zb[1]), row(zc[0]), row(zc[1]),
            full(WnT), full(WnB), full(bn2),
            full(WnlT), full(WnlBA), full(WnlBB), full(WnlBC), full(bnl2),
        ],
        out_specs=[
            pl.BlockSpec((RB, D), lambda i: (i, 0)),
            pl.BlockSpec((RB, DW), lambda i: (i, 0)),
        ],
        out_shape=[
            jax.ShapeDtypeStruct((N, D), F32),
            jax.ShapeDtypeStruct((N, DW), F32),
        ],
    )(x, w2v, zf[0], zf[1], za[0], za[1], zb[0], zb[1], zc[0], zc[1],
      WnT, WnB, bn2, WnlT, WnlBA, WnlBB, WnlBC, bnl2)


def _p1_edges(src_tab, dst_tab, cedge, src_idx, dst_idx, wa, wal, params, NP):
    N, SRCW = src_tab.shape
    DSTW = dst_tab.shape[1]
    E = src_idx.shape[0]
    D = cedge.shape[1]
    EPW = E // NW
    NCH = EPW // CK
    RPT = NP // NS
    RZB = RPT // 5
    NJA = D // 16
    NJL = (SRCW - 2 * D) // 16

    @functools.partial(
        pl.kernel,
        out_type=[
            jax.ShapeDtypeStruct((E, 16), F32),
            jax.ShapeDtypeStruct((E, D), F32),
            jax.ShapeDtypeStruct((NC, NP, 16), F32),
        ],
        mesh=_mesh(),
        compiler_params=_SC_PARAMS,
        scratch_types=[
            pltpu.VMEM((CK,), jnp.int32),
            pltpu.VMEM((CK,), jnp.int32),
            pltpu.VMEM((CK, SRCW), F32),
            pltpu.VMEM((CK, DSTW), F32),
            pltpu.VMEM((CK, D), F32),
            pltpu.VMEM((CK, D), F32),
            pltpu.VMEM((CK, 16), F32),
            pltpu.VMEM((CK * 16,), F32),
            pltpu.VMEM((CK * 16,), F32),
            pltpu.VMEM((D,), F32),
            pltpu.VMEM((SRCW - 2 * D,), F32),
            pltpu.VMEM((16,), F32),
            pltpu.VMEM((RZB, 16), F32),
            pltpu.VMEM_SHARED((NP, 16), F32),
            pltpu.SemaphoreType.DMA,
            pltpu.SemaphoreType.DMA,
        ],
    )
    def k(src_tab_h, dst_tab_h, c_h, sidx_h, didx_h, wa_h, wal_h, par_h,
          w16_h, zraw_h, s16_h,
          sidx_v, didx_v, srow_v, drow_v, c_v, zraw_v, w16_v, aa_v, al_v,
          wa_v, wal_v, par_v, zb_v, s16_sh, sem1, sem2):
        cid = lax.axis_index("c")
        sid = lax.axis_index("s")
        wid = sid * NC + cid
        pltpu.sync_copy(wa_h, wa_v)
        pltpu.sync_copy(wal_h, wal_v)
        pltpu.sync_copy(par_h, par_v)
        parv = par_v[...]
        ba = parv[0]
        bal = parv[1]

        def zrow(i, carry):
            zb_v[i, :] = jnp.zeros((16,), F32)
            return carry
        lax.fori_loop(0, RZB, zrow, 0)
        for i in range(5):
            pltpu.sync_copy(zb_v, s16_sh.at[pl.ds(sid * RPT + i * RZB, RZB)])

        def zw(e, carry):
            w16_v[e, :] = jnp.zeros((16,), F32)
            return carry
        lax.fori_loop(0, CK, zw, 0)
        plsc.subcore_barrier()

        ii = lax.iota(jnp.int32, 16)
        wa_c = [wa_v[pl.ds(16 * j, 16)] for j in range(NJA)]
        wal_c = [wal_v[pl.ds(16 * j, 16)] for j in range(NJL)]

        def chunk_body(g, carry):
            base = wid * EPW + g * CK
            pltpu.sync_copy(sidx_h.at[pl.ds(base, CK)], sidx_v)
            pltpu.sync_copy(didx_h.at[pl.ds(base, CK)], didx_v)
            pltpu.async_copy(src_tab_h.at[sidx_v], srow_v, sem1).wait()
            pltpu.async_copy(dst_tab_h.at[didx_v], drow_v, sem2).wait()
            pltpu.sync_copy(c_h.at[pl.ds(base, CK)], c_v)

            def edge_body(e, c2):
                acc_a = jnp.zeros((16,), F32)
                for j in range(NJA):
                    sl = pl.ds(16 * j, 16)
                    ef = jnp.maximum(
                        srow_v[e, sl] + drow_v[e, sl] + c_v[e, sl], 0.0)
                    zraw_v[e, sl] = srow_v[e, pl.ds(D + 16 * j, 16)] + ef
                    acc_a = acc_a + ef * wa_c[j]
                acc_l = jnp.zeros((16,), F32)
                for j in range(NJL):
                    r = jnp.maximum(
                        srow_v[e, pl.ds(2 * D + 16 * j, 16)]
                        + drow_v[e, pl.ds(D + 16 * j, 16)], 0.0)
                    acc_l = acc_l + r * wal_c[j]
                aa_v[pl.ds(16 * e, 16)] = acc_a
                al_v[pl.ds(16 * e, 16)] = acc_l
                return c2
            lax.fori_loop(0, CK, edge_body, 0)

            def grp_body(l, c2):
                base16 = 256 * l
                suma = jnp.zeros((16,), F32)
                suml = jnp.zeros((16,), F32)
                for c in range(16):
                    suma = suma + plsc.load_gather(
                        aa_v, [base16 + ii * 16 + c])
                    suml = suml + plsc.load_gather(
                        al_v, [base16 + ii * 16 + c])
                w = jnp.exp(jnp.maximum(suma + ba, 0.0))
                wl = jnp.exp(jnp.maximum(suml + bal, 0.0))
                rows = 16 * l + ii
                zl = ii * 0
                plsc.store_scatter(w16_v, [rows, zl], w)
                plsc.store_scatter(w16_v, [rows, zl + 1], wl)
                return c2
            lax.fori_loop(0, CK // 16, grp_body, 0)
            pltpu.sync_copy(w16_v, w16_h.at[pl.ds(base, CK)])
            pltpu.sync_copy(zraw_v, zraw_h.at[pl.ds(base, CK)])
            pltpu.sync_copy(w16_v, s16_sh.at[didx_v], add=True)
            return carry
        lax.fori_loop(0, NCH, chunk_body, 0)

        plsc.subcore_barrier()
        for i in range(5):
            sl = pl.ds(sid * RPT + i * RZB, RZB)
            pltpu.sync_copy(s16_sh.at[sl], s16_h.at[cid, sl])

    return k(src_tab, dst_tab, cedge, src_idx, dst_idx, wa, wal, params)


def _p2_zf(zraw, w16f, dst_idx, recip):
    E, D = zraw.shape
    NP = recip.shape[1]
    EPW = E // NW
    NCH = EPW // CK
    RPT = NP // NS
    RZB = RPT // 5
    NJA = D // 16

    @functools.partial(
        pl.kernel,
        out_type=jax.ShapeDtypeStruct((NC, NP, D), F32),
        mesh=_mesh(),
        compiler_params=_SC_PARAMS,
        scratch_types=[
            pltpu.VMEM((CK,), jnp.int32),
            pltpu.VMEM((CK, D), F32),
            pltpu.VMEM((CK * 16,), F32),
            pltpu.VMEM((NP,), F32),
            pltpu.VMEM((RZB, D), F32),
            pltpu.VMEM_SHARED((NP, D), F32),
        ],
    )
    def k(zraw_h, w16f_h, didx_h, recip_h, out_h,
          didx_v, zrow_v, w16_v, rec_v, zb_v, z_sh):
        cid = lax.axis_index("c")
        sid = lax.axis_index("s")
        wid = sid * NC + cid
        pltpu.sync_copy(recip_h.at[0], rec_v)

        def zrow(i, carry):
            for j in range(NJA):
                zb_v[i, pl.ds(16 * j, 16)] = jnp.zeros((16,), F32)
            return carry
        lax.fori_loop(0, RZB, zrow, 0)
        for i in range(5):
            pltpu.sync_copy(zb_v, z_sh.at[pl.ds(sid * RPT + i * RZB, RZB)])
        plsc.subcore_barrier()

        ii = lax.iota(jnp.int32, 16)

        def chunk_body(g, carry):
            base = wid * EPW + g * CK
            pltpu.sync_copy(didx_h.at[pl.ds(base, CK)], didx_v)
            pltpu.sync_copy(zraw_h.at[pl.ds(base, CK)], zrow_v)
            pltpu.sync_copy(w16f_h.at[pl.ds(base * 16, CK * 16)], w16_v)

            def grp(l, c2):
                d16 = didx_v[pl.ds(16 * l, 16)]
                rec = plsc.load_gather(rec_v, [d16])
                wv = plsc.load_gather(w16_v, [256 * l + ii * 16])
                alpha = wv * rec
                for i in range(16):
                    a = alpha[i]
                    e = 16 * l + i
                    for j in range(NJA):
                        sl = pl.ds(16 * j, 16)
                        zrow_v[e, sl] = zrow_v[e, sl] * a
                return c2
            lax.fori_loop(0, CK // 16, grp, 0)
            pltpu.sync_copy(zrow_v, z_sh.at[didx_v], add=True)
            return carry
        lax.fori_loop(0, NCH, chunk_body, 0)

        plsc.subcore_barrier()
        for i in range(5):
            sl = pl.ds(sid * RPT + i * RZB, RZB)
            pltpu.sync_copy(z_sh.at[sl], out_h.at[cid, sl])

    return k(zraw, w16f, dst_idx, recip)


def _p3_lang(tab, w16f, src_idx, dst_idx, recip):
    N, W = tab.shape
    NP = recip.shape[1]
    E = src_idx.shape[0]
    EPW = E // NW
    NCH = EPW // CK
    RPT = NP // NS
    RZB = RPT // 5
    NJ = W // 16

    @functools.partial(
        pl.kernel,
        out_type=jax.ShapeDtypeStruct((NC, NP, W), F32),
        mesh=_mesh(),
        compiler_params=_SC_PARAMS,
        scratch_types=[
            pltpu.VMEM((CK,), jnp.int32),
            pltpu.VMEM((CK,), jnp.int32),
            pltpu.VMEM((CK, W), F32),
            pltpu.VMEM((CK * 16,), F32),
            pltpu.VMEM((NP,), F32),
            pltpu.VMEM((RZB, W), F32),
            pltpu.VMEM_SHARED((NP, W), F32),
            pltpu.SemaphoreType.DMA,
        ],
    )
    def k(tab_h, w16f_h, sidx_h, didx_h, recip_h, out_h,
          sidx_v, didx_v, row_v, w16_v, rec_v, zb_v, z_sh, sem1):
        cid = lax.axis_index("c")
        sid = lax.axis_index("s")
        wid = sid * NC + cid
        pltpu.sync_copy(recip_h.at[1], rec_v)

        def zrow(i, carry):
            for j in range(NJ):
                zb_v[i, pl.ds(16 * j, 16)] = jnp.zeros((16,), F32)
            return carry
        lax.fori_loop(0, RZB, zrow, 0)
        for i in range(5):
            pltpu.sync_copy(zb_v, z_sh.at[pl.ds(sid * RPT + i * RZB, RZB)])
        plsc.subcore_barrier()

        ii = lax.iota(jnp.int32, 16)

        def chunk_body(g, carry):
            base = wid * EPW + g * CK
            pltpu.sync_copy(sidx_h.at[pl.ds(base, CK)], sidx_v)
            pltpu.sync_copy(didx_h.at[pl.ds(base, CK)], didx_v)
            pltpu.async_copy(tab_h.at[sidx_v], row_v, sem1).wait()
            pltpu.sync_copy(w16f_h.at[pl.ds(base * 16, CK * 16)], w16_v)

            def grp(l, c2):
                d16 = didx_v[pl.ds(16 * l, 16)]
                rec = plsc.load_gather(rec_v, [d16])
                wv = plsc.load_gather(w16_v, [256 * l + ii * 16 + 1])
                alpha = wv * rec
                for i in range(16):
                    a = alpha[i]
                    e = 16 * l + i
                    for j in range(NJ):
                        sl = pl.ds(16 * j, 16)
                        row_v[e, sl] = row_v[e, sl] * a
                return c2
            lax.fori_loop(0, CK // 16, grp, 0)
            pltpu.sync_copy(row_v, z_sh.at[didx_v], add=True)
            return carry
        lax.fori_loop(0, NCH, chunk_body, 0)

        plsc.subcore_barrier()
        for i in range(5):
            sl = pl.ds(sid * RPT + i * RZB, RZB)
            pltpu.sync_copy(z_sh.at[sl], out_h.at[cid, sl])

    return k(tab, w16f, src_idx, dst_idx, recip)


def kernel(x, word2vec, s_f, edge_index, W_e, b_e, W_el, b_el,
           W_a, b_a, W_al, b_al, W_n, b_n, W_nl, b_nl):
    N, D = x.shape
    DW = word2vec.shape[1]
    E, DS = s_f.shape
    DWP = DW + 4
    LA = 128
    LB = 128
    LC = DWP - LA - LB

    src_idx = edge_index[0].astype(jnp.int32)
    dst_idx = edge_index[1].astype(jnp.int32)

    WeT = W_e[0:D]
    WeM = W_e[D:D + DS]
    WeB = W_e[D + DS:]
    WelT = jnp.pad(W_el[0:DW], ((0, 0), (0, DWP - DW)))
    WelB = jnp.pad(W_el[DW:], ((0, 0), (0, DWP - DW)))
    belp = jnp.pad(b_el, (0, DWP - DW)).reshape(1, DWP)
    be2 = b_e.reshape(1, D)
    wa = W_a[:, 0]
    wal = jnp.pad(W_al[:, 0], (0, DWP - DW))
    params = jnp.concatenate(
        [b_a.astype(F32), b_al.astype(F32), jnp.zeros((14,), F32)])
    w2vA = word2vec[:, 0:LA]
    w2vB = word2vec[:, LA:LA + LB]
    w2vC = jnp.pad(word2vec[:, LA + LB:], ((0, 0), (0, DWP - DW)))
    WnT = W_n[0:D]
    WnB = W_n[D:]
    bn2 = b_n.reshape(1, D)
    WnlT = W_nl[0:DW]
    WnlBA = W_nl[DW:DW + LA]
    WnlBB = W_nl[DW + LA:DW + LA + LB]
    WnlBC = jnp.pad(W_nl[DW + LA + LB:], ((0, DWP - DW), (0, 0)))
    bnl2 = b_nl.reshape(1, DW)

    NP = 10240

    src_tab, dst_tab = _t1_tables(x, word2vec, WeT, WeB, WelT, WelB, belp)
    cedge = _t2_cedge(s_f, WeM, be2)
    w16, zraw, s16 = _p1_edges(
        src_tab, dst_tab, cedge, src_idx, dst_idx, wa, wal, params, NP)
    recip = _t3_recip(s16)
    w16f = w16.reshape(E * 16)
    zf = _p2_zf(zraw, w16f, dst_idx, recip)
    za = _p3_lang(w2vA, w16f, src_idx, dst_idx, recip)
    zb = _p3_lang(w2vB, w16f, src_idx, dst_idx, recip)
    zc = _p3_lang(w2vC, w16f, src_idx, dst_idx, recip)
    o1, o2 = _t4_node(
        x, word2vec, zf, za, zb, zc,
        WnT, WnB, bn2, WnlT, WnlBA, WnlBB, WnlBC, bnl2)
    return (o1, o2)


# speedup vs baseline: 4.8492x; 1.2733x over previous
import functools

import jax
import jax.numpy as jnp
from jax import lax
from jax.experimental import pallas as pl
from jax.experimental.pallas import tpu as pltpu
from jax.experimental.pallas import tpu_sc as plsc

F32 = jnp.float32
NC = 2
NS = 16
NW = NC * NS
CK = 80


def _mesh():
    return plsc.VectorSubcoreMesh(
        core_axis_name="c", subcore_axis_name="s", num_cores=NC, num_subcores=NS)


_SC_PARAMS = pltpu.CompilerParams(
    needs_layout_passes=False, use_tc_tiling_on_sc=False)


def _t1_tables(x, w2v, WeT, WeB, WelT, WelB, belp):
    N, D = x.shape
    DWP = WelT.shape[1]
    NB = 5
    RB = N // NB

    def body(x_ref, wv_ref, wet, web, welt, welb, bel,
             ax_ref, b_ref, p_ref, q_ref):
        xb = x_ref[...]
        wv = wv_ref[...]
        ax_ref[:, 0:D] = jnp.dot(xb, wet[...], preferred_element_type=F32)
        ax_ref[:, D:2 * D] = xb
        b_ref[...] = jnp.dot(xb, web[...], preferred_element_type=F32)
        p_ref[...] = (
            jnp.dot(wv, welt[...], preferred_element_type=F32) + bel[...])
        q_ref[...] = jnp.dot(wv, welb[...], preferred_element_type=F32)

    return pl.pallas_call(
        body,
        grid=(NB,),
        in_specs=[
            pl.BlockSpec((RB, x.shape[1]), lambda i: (i, 0)),
            pl.BlockSpec((RB, w2v.shape[1]), lambda i: (i, 0)),
            pl.BlockSpec(WeT.shape, lambda i: (0, 0)),
            pl.BlockSpec(WeB.shape, lambda i: (0, 0)),
            pl.BlockSpec(WelT.shape, lambda i: (0, 0)),
            pl.BlockSpec(WelB.shape, lambda i: (0, 0)),
            pl.BlockSpec(belp.shape, lambda i: (0, 0)),
        ],
        out_specs=[
            pl.BlockSpec((RB, 2 * D), lambda i: (i, 0)),
            pl.BlockSpec((RB, D), lambda i: (i, 0)),
            pl.BlockSpec((RB, DWP), lambda i: (i, 0)),
            pl.BlockSpec((RB, DWP), lambda i: (i, 0)),
        ],
        out_shape=[
            jax.ShapeDtypeStruct((N, 2 * D), F32),
            jax.ShapeDtypeStruct((N, D), F32),
            jax.ShapeDtypeStruct((N, DWP), F32),
            jax.ShapeDtypeStruct((N, DWP), F32),
        ],
    )(x, w2v, WeT, WeB, WelT, WelB, belp)


def _t2_cedge(s_f, WeM, be2):
    E, DS = s_f.shape
    D = WeM.shape[1]
    EB = 8000
    NB = E // EB

    def body(sf_ref, w_ref, b_ref, c_ref):
        c_ref[...] = (
            jnp.dot(sf_ref[...], w_ref[...], preferred_element_type=F32)
            + b_ref[...])

    return pl.pallas_call(
        body,
        grid=(NB,),
        in_specs=[
            pl.BlockSpec((EB, DS), lambda i: (i, 0)),
            pl.BlockSpec(WeM.shape, lambda i: (0, 0)),
            pl.BlockSpec(be2.shape, lambda i: (0, 0)),
        ],
        out_specs=pl.BlockSpec((EB, D), lambda i: (i, 0)),
        out_shape=jax.ShapeDtypeStruct((E, D), F32),
    )(s_f, WeM, be2)


def _t3_recip(s16a, s16b):
    _, NP, L = s16a.shape

    def body(sa_ref, sb_ref, o_ref):
        sa = sa_ref[0] + sa_ref[1]
        sb = sb_ref[0] + sb_ref[1]
        lane = lax.broadcasted_iota(jnp.int32, (NP, L), 1)
        tot = jnp.sum(jnp.where(lane == 0, sa, 0.0), axis=1)
        totl = jnp.sum(jnp.where(lane == 1, sb, 0.0), axis=1)
        r0 = (1.0 / (tot + 1e-9)).reshape(1, NP)
        r1 = (1.0 / (totl + 1e-9)).reshape(1, NP)
        o_ref[...] = jnp.concatenate([r0, r1], axis=0)

    return pl.pallas_call(
        body, out_shape=jax.ShapeDtypeStruct((2, NP), F32))(s16a, s16b)


def _t4_node(x, w2v, zfA, zfB, za, zb, zc,
             WnT, WnBA, WnBB, bn2, WnlT, WnlBA, WnlBB, WnlBC, bnl2):
    N, D = x.shape
    DW = w2v.shape[1]
    NB = 5
    RB = N // NB

    def body(x_ref, wv_ref, zA0r, zA1r, zB0r, zB1r,
             za0r, za1r, zb0r, zb1r, zc0r, zc1r,
             wnt, wnba, wnbb, bn, wnlt, wnlba, wnlbb, wnlbc, bnl,
             o1_ref, o2_ref):
        zA = zA0r[...] + zA1r[...]
        zB = zB0r[...] + zB1r[...]
        o1_ref[...] = jnp.maximum(
            jnp.dot(x_ref[...], wnt[...], preferred_element_type=F32)
            + jnp.dot(zA, wnba[...], preferred_element_type=F32)
            + jnp.dot(zB, wnbb[...], preferred_element_type=F32)
            + bn[...], 0.0)
        zav = za0r[...] + za1r[...]
        zbv = zb0r[...] + zb1r[...]
        zcv = zc0r[...] + zc1r[...]
        o2_ref[...] = jnp.maximum(
            jnp.dot(wv_ref[...], wnlt[...], preferred_element_type=F32)
            + jnp.dot(zav, wnlba[...], preferred_element_type=F32)
            + jnp.dot(zbv, wnlbb[...], preferred_element_type=F32)
            + jnp.dot(zcv, wnlbc[...], preferred_element_type=F32)
            + bnl[...], 0.0)

    row = lambda a: pl.BlockSpec((RB, a.shape[-1]), lambda i: (i, 0))
    full = lambda a: pl.BlockSpec(a.shape, lambda i: tuple(0 for _ in a.shape))
    return pl.pallas_call(
        body,
        grid=(NB,),
        in_specs=[
            row(x), row(w2v),
            row(zfA[0]), row(zfA[1]), row(zfB[0]), row(zfB[1]),
            row(za[0]), row(za[1]),
            row(zb[0]), row(zb[1]), row(zc[0]), row(zc[1]),
            full(WnT), full(WnBA), full(WnBB), full(bn2),
            full(WnlT), full(WnlBA), full(WnlBB), full(WnlBC), full(bnl2),
        ],
        out_specs=[
            pl.BlockSpec((RB, D), lambda i: (i, 0)),
            pl.BlockSpec((RB, DW), lambda i: (i, 0)),
        ],
        out_shape=[
            jax.ShapeDtypeStruct((N, D), F32),
            jax.ShapeDtypeStruct((N, DW), F32),
        ],
    )(x, w2v, zfA[0], zfA[1], zfB[0], zfB[1],
      za[0], za[1], zb[0], zb[1], zc[0], zc[1],
      WnT, WnBA, WnBB, bn2, WnlT, WnlBA, WnlBB, WnlBC, bnl2)


def _zero_shared(sh, zb_v, sid, RPT, RZB, NJ):
    def zrow(i, carry):
        for j in range(NJ):
            zb_v[i, pl.ds(16 * j, 16)] = jnp.zeros((16,), F32)
        return carry
    lax.fori_loop(0, RZB, zrow, 0)
    for i in range(RPT // RZB):
        pltpu.sync_copy(zb_v, sh.at[pl.ds(sid * RPT + i * RZB, RZB)])


def _dump_shared(sh, out_h, cid, sid, RPT, RZB):
    for i in range(RPT // RZB):
        sl = pl.ds(sid * RPT + i * RZB, RZB)
        pltpu.sync_copy(sh.at[sl], out_h.at[cid, sl])


def _run_chunks(NCH, issue, step):
    issue(0, 0)
    issue(1, 1)

    def pair(p, carry):
        step(0, 2 * p)
        step(1, 2 * p + 1)
        return carry
    lax.fori_loop(0, NCH // 2, pair, 0)
    if NCH % 2:
        step(0, NCH - 1)


def _p1_feat(ax_tab, b_tab, cedge, src_idx, dst_idx, wa, params, NP):
    E = src_idx.shape[0]
    D = b_tab.shape[1]
    EPW = E // NW
    NCH = EPW // CK
    RPT = NP // NS
    RZB = RPT // 5
    NJA = D // 16

    @functools.partial(
        pl.kernel,
        out_type=[
            jax.ShapeDtypeStruct((E, 16), F32),
            jax.ShapeDtypeStruct((E, D // 2), F32),
            jax.ShapeDtypeStruct((E, D // 2), F32),
            jax.ShapeDtypeStruct((NC, NP, 16), F32),
        ],
        mesh=_mesh(),
        compiler_params=_SC_PARAMS,
        scratch_types=[
            pltpu.VMEM((2, CK), jnp.int32),
            pltpu.VMEM((2, CK), jnp.int32),
            pltpu.VMEM((2, CK, 2 * 128), F32),
            pltpu.VMEM((2, CK, 128), F32),
            pltpu.VMEM((2, CK, 128), F32),
            pltpu.VMEM((CK, 64), F32),
            pltpu.VMEM((CK, 64), F32),
            pltpu.VMEM((CK, 16), F32),
            pltpu.VMEM((CK * 16,), F32),
            pltpu.VMEM((128,), F32),
            pltpu.VMEM((16,), F32),
            pltpu.VMEM((128, 16), F32),
            pltpu.VMEM_SHARED((10240, 16), F32),
            pltpu.SemaphoreType.DMA,
            pltpu.SemaphoreType.DMA,
            pltpu.SemaphoreType.DMA,
            pltpu.SemaphoreType.DMA,
            pltpu.SemaphoreType.DMA,
            pltpu.SemaphoreType.DMA,
        ],
    )
    def k(ax_h, b_h, c_h, sidx_h, didx_h, wa_h, par_h,
          w16_h, zrawA_h, zrawB_h, s16_h,
          sidx_v, didx_v, srow_v, drow_v, c_v, zrawA_v, zrawB_v, w16_v, aa_v,
          wa_v, par_v, zb_v, s16_sh,
          semS0, semS1, semD0, semD1, semC0, semC1):
        semS = (semS0, semS1)
        semD = (semD0, semD1)
        semC = (semC0, semC1)
        cid = lax.axis_index("c")
        sid = lax.axis_index("s")
        wid = sid * NC + cid
        pltpu.sync_copy(wa_h, wa_v)
        pltpu.sync_copy(par_h, par_v)
        parv = par_v[...]
        ba = parv[0]
        _zero_shared(s16_sh, zb_v, sid, RPT, RZB, 1)

        def zw(e, carry):
            w16_v[e, :] = jnp.zeros((16,), F32)
            return carry
        lax.fori_loop(0, CK, zw, 0)
        plsc.subcore_barrier()

        ii = lax.iota(jnp.int32, 16)
        wa_c = [wa_v[pl.ds(16 * j, 16)] for j in range(NJA)]

        def issue(b, g):
            base = wid * EPW + g * CK
            pltpu.sync_copy(sidx_h.at[pl.ds(base, CK)], sidx_v.at[b])
            pltpu.sync_copy(didx_h.at[pl.ds(base, CK)], didx_v.at[b])
            pltpu.async_copy(ax_h.at[sidx_v.at[b]], srow_v.at[b], semS[b])
            pltpu.async_copy(b_h.at[didx_v.at[b]], drow_v.at[b], semD[b])
            pltpu.async_copy(c_h.at[pl.ds(base, CK)], c_v.at[b], semC[b])

        def step(b, g):
            pltpu.make_async_copy(
                ax_h.at[sidx_v.at[b]], srow_v.at[b], semS[b]).wait()
            pltpu.make_async_copy(
                b_h.at[didx_v.at[b]], drow_v.at[b], semD[b]).wait()
            pltpu.make_async_copy(
                c_h.at[pl.ds(0, CK)], c_v.at[b], semC[b]).wait()
            base = wid * EPW + g * CK

            def edge_body(e, c2):
                acc_a = jnp.zeros((16,), F32)
                for j in range(NJA):
                    sl = pl.ds(16 * j, 16)
                    ef = jnp.maximum(
                        srow_v[b, e, sl] + drow_v[b, e, sl] + c_v[b, e, sl],
                        0.0)
                    zr = srow_v[b, e, pl.ds(D + 16 * j, 16)] + ef
                    if j < NJA // 2:
                        zrawA_v[e, pl.ds(16 * j, 16)] = zr
                    else:
                        zrawB_v[e, pl.ds(16 * (j - NJA // 2), 16)] = zr
                    acc_a = acc_a + ef * wa_c[j]
                aa_v[pl.ds(16 * e, 16)] = acc_a
                return c2
            lax.fori_loop(0, CK, edge_body, 0)

            def grp_body(l, c2):
                base16 = 256 * l
                suma = jnp.zeros((16,), F32)
                for c in range(16):
                    suma = suma + plsc.load_gather(
                        aa_v, [base16 + ii * 16 + c])
                w = jnp.exp(jnp.maximum(suma + ba, 0.0))
                plsc.store_scatter(w16_v, [16 * l + ii, ii * 0], w)
                return c2
            lax.fori_loop(0, CK // 16, grp_body, 0)

            pltpu.sync_copy(w16_v, w16_h.at[pl.ds(base, CK)])
            pltpu.sync_copy(zrawA_v, zrawA_h.at[pl.ds(base, CK)])
            pltpu.sync_copy(zrawB_v, zrawB_h.at[pl.ds(base, CK)])
            pltpu.sync_copy(w16_v, s16_sh.at[didx_v.at[b]], add=True)

            @pl.when(g + 2 < NCH)
            def _():
                issue(b, g + 2)

        _run_chunks(NCH, issue, step)
        plsc.subcore_barrier()
        _dump_shared(s16_sh, s16_h, cid, sid, RPT, RZB)

    return k(ax_tab, b_tab, cedge, src_idx, dst_idx, wa, params)


def _p1_lang(p_tab, q_tab, src_idx, dst_idx, wal, params, NP):
    DWP = p_tab.shape[1]
    E = src_idx.shape[0]
    EPW = E // NW
    NCH = EPW // CK
    RPT = NP // NS
    RZB = RPT // 5
    NJL = DWP // 16

    @functools.partial(
        pl.kernel,
        out_type=[
            jax.ShapeDtypeStruct((E, 16), F32),
            jax.ShapeDtypeStruct((NC, NP, 16), F32),
        ],
        mesh=_mesh(),
        compiler_params=_SC_PARAMS,
        scratch_types=[
            pltpu.VMEM((2, CK), jnp.int32),
            pltpu.VMEM((2, CK), jnp.int32),
            pltpu.VMEM((2, CK, 304), F32),
            pltpu.VMEM((2, CK, 304), F32),
            pltpu.VMEM((CK, 16), F32),
            pltpu.VMEM((CK * 16,), F32),
            pltpu.VMEM((304,), F32),
            pltpu.VMEM((16,), F32),
            pltpu.VMEM((128, 16), F32),
            pltpu.VMEM_SHARED((10240, 16), F32),
            pltpu.SemaphoreType.DMA,
            pltpu.SemaphoreType.DMA,
            pltpu.SemaphoreType.DMA,
            pltpu.SemaphoreType.DMA,
        ],
    )
    def k(p_h, q_h, sidx_h, didx_h, wal_h, par_h,
          w16_h, s16_h,
          sidx_v, didx_v, srow_v, drow_v, w16_v, al_v,
          wal_v, par_v, zb_v, s16_sh,
          semS0, semS1, semD0, semD1):
        semS = (semS0, semS1)
        semD = (semD0, semD1)
        cid = lax.axis_index("c")
        sid = lax.axis_index("s")
        wid = sid * NC + cid
        pltpu.sync_copy(wal_h, wal_v)
        pltpu.sync_copy(par_h, par_v)
        parv = par_v[...]
        bal = parv[1]
        _zero_shared(s16_sh, zb_v, sid, RPT, RZB, 1)

        def zw(e, carry):
            w16_v[e, :] = jnp.zeros((16,), F32)
            return carry
        lax.fori_loop(0, CK, zw, 0)
        plsc.subcore_barrier()

        ii = lax.iota(jnp.int32, 16)
        wal_c = [wal_v[pl.ds(16 * j, 16)] for j in range(NJL)]

        def issue(b, g):
            base = wid * EPW + g * CK
            pltpu.sync_copy(sidx_h.at[pl.ds(base, CK)], sidx_v.at[b])
            pltpu.sync_copy(didx_h.at[pl.ds(base, CK)], didx_v.at[b])
            pltpu.async_copy(p_h.at[sidx_v.at[b]], srow_v.at[b], semS[b])
            pltpu.async_copy(q_h.at[didx_v.at[b]], drow_v.at[b], semD[b])

        def step(b, g):
            pltpu.make_async_copy(
                p_h.at[sidx_v.at[b]], srow_v.at[b], semS[b]).wait()
            pltpu.make_async_copy(
                q_h.at[didx_v.at[b]], drow_v.at[b], semD[b]).wait()
            base = wid * EPW + g * CK

            def edge_body(e, c2):
                acc_l = jnp.zeros((16,), F32)
                for j in range(NJL):
                    sl = pl.ds(16 * j, 16)
                    r = jnp.maximum(srow_v[b, e, sl] + drow_v[b, e, sl], 0.0)
                    acc_l = acc_l + r * wal_c[j]
                al_v[pl.ds(16 * e, 16)] = acc_l
                return c2
            lax.fori_loop(0, CK, edge_body, 0)

            def grp_body(l, c2):
                base16 = 256 * l
                suml = jnp.zeros((16,), F32)
                for c in range(16):
                    suml = suml + plsc.load_gather(
                        al_v, [base16 + ii * 16 + c])
                wl = jnp.exp(jnp.maximum(suml + bal, 0.0))
                plsc.store_scatter(w16_v, [16 * l + ii, ii * 0 + 1], wl)
                return c2
            lax.fori_loop(0, CK // 16, grp_body, 0)

            pltpu.sync_copy(w16_v, w16_h.at[pl.ds(base, CK)])
            pltpu.sync_copy(w16_v, s16_sh.at[didx_v.at[b]], add=True)

            @pl.when(g + 2 < NCH)
            def _():
                issue(b, g + 2)

        _run_chunks(NCH, issue, step)
        plsc.subcore_barrier()
        _dump_shared(s16_sh, s16_h, cid, sid, RPT, RZB)

    return k(p_tab, q_tab, src_idx, dst_idx, wal, params)


def _p2_zf(zraw, w16f, dst_idx, recip, W):
    E = zraw.shape[0]
    NP = recip.shape[1]
    EPW = E // NW
    NCH = EPW // CK
    RPT = NP // NS
    RZB = RPT // 5
    NJA = W // 16

    @functools.partial(
        pl.kernel,
        out_type=jax.ShapeDtypeStruct((NC, NP, W), F32),
        mesh=_mesh(),
        compiler_params=_SC_PARAMS,
        scratch_types=[
            pltpu.VMEM((2, CK), jnp.int32),
            pltpu.VMEM((2, CK, W), F32),
            pltpu.VMEM((2, CK * 16), F32),
            pltpu.VMEM((10240,), F32),
            pltpu.VMEM((128, W), F32),
            pltpu.VMEM_SHARED((10240, W), F32),
            pltpu.SemaphoreType.DMA,
            pltpu.SemaphoreType.DMA,
            pltpu.SemaphoreType.DMA,
            pltpu.SemaphoreType.DMA,
        ],
    )
    def k(zraw_h, w16f_h, didx_h, recip_h, out_h,
          didx_v, zrow_v, w16_v, rec_v, zb_v, z_sh,
          semZ0, semZ1, semW0, semW1):
        semZ = (semZ0, semZ1)
        semW = (semW0, semW1)
        cid = lax.axis_index("c")
        sid = lax.axis_index("s")
        wid = sid * NC + cid
        pltpu.sync_copy(recip_h.at[0], rec_v)
        _zero_shared(z_sh, zb_v, sid, RPT, RZB, NJA)
        plsc.subcore_barrier()

        ii = lax.iota(jnp.int32, 16)

        def issue(b, g):
            base = wid * EPW + g * CK
            pltpu.sync_copy(didx_h.at[pl.ds(base, CK)], didx_v.at[b])
            pltpu.async_copy(
                zraw_h.at[pl.ds(base, CK)], zrow_v.at[b], semZ[b])
            pltpu.async_copy(
                w16f_h.at[pl.ds(base * 16, CK * 16)], w16_v.at[b], semW[b])

        def step(b, g):
            pltpu.make_async_copy(
                zraw_h.at[pl.ds(0, CK)], zrow_v.at[b], semZ[b]).wait()
            pltpu.make_async_copy(
                w16f_h.at[pl.ds(0, CK * 16)], w16_v.at[b], semW[b]).wait()

            def grp(l, c2):
                d16 = didx_v[b, pl.ds(16 * l, 16)]
                rec = plsc.load_gather(rec_v, [d16])
                wv = plsc.load_gather(w16_v.at[b], [256 * l + ii * 16])
                alpha = wv * rec
                for i in range(16):
                    a = alpha[i]
                    e = 16 * l + i
                    for j in range(NJA):
                        sl = pl.ds(16 * j, 16)
                        zrow_v[b, e, sl] = zrow_v[b, e, sl] * a
                return c2
            lax.fori_loop(0, CK // 16, grp, 0)
            pltpu.sync_copy(zrow_v.at[b], z_sh.at[didx_v.at[b]], add=True)

            @pl.when(g + 2 < NCH)
            def _():
                issue(b, g + 2)

        _run_chunks(NCH, issue, step)
        plsc.subcore_barrier()
        _dump_shared(z_sh, out_h, cid, sid, RPT, RZB)

    return k(zraw, w16f, dst_idx, recip)


def _p3_lang(tab, w16f, src_idx, dst_idx, recip, W):
    NP = recip.shape[1]
    E = src_idx.shape[0]
    EPW = E // NW
    NCH = EPW // CK
    RPT = NP // NS
    RZB = RPT // 5
    NJ = W // 16

    @functools.partial(
        pl.kernel,
        out_type=jax.ShapeDtypeStruct((NC, NP, W), F32),
        mesh=_mesh(),
        compiler_params=_SC_PARAMS,
        scratch_types=[
            pltpu.VMEM((2, CK), jnp.int32),
            pltpu.VMEM((2, CK), jnp.int32),
            pltpu.VMEM((2, CK, W), F32),
            pltpu.VMEM((2, CK * 16), F32),
            pltpu.VMEM((10240,), F32),
            pltpu.VMEM((128, W), F32),
            pltpu.VMEM_SHARED((10240, W), F32),
            pltpu.SemaphoreType.DMA,
            pltpu.SemaphoreType.DMA,
            pltpu.SemaphoreType.DMA,
            pltpu.SemaphoreType.DMA,
        ],
    )
    def k(tab_h, w16f_h, sidx_h, didx_h, recip_h, out_h,
          sidx_v, didx_v, row_v, w16_v, rec_v, zb_v, z_sh,
          semR0, semR1, semW0, semW1):
        semR = (semR0, semR1)
        semW = (semW0, semW1)
        cid = lax.axis_index("c")
        sid = lax.axis_index("s")
        wid = sid * NC + cid
        pltpu.sync_copy(recip_h.at[1], rec_v)
        _zero_shared(z_sh, zb_v, sid, RPT, RZB, NJ)
        plsc.subcore_barrier()

        ii = lax.iota(jnp.int32, 16)

        def issue(b, g):
            base = wid * EPW + g * CK
            pltpu.sync_copy(sidx_h.at[pl.ds(base, CK)], sidx_v.at[b])
            pltpu.sync_copy(didx_h.at[pl.ds(base, CK)], didx_v.at[b])
            pltpu.async_copy(tab_h.at[sidx_v.at[b]], row_v.at[b], semR[b])
            pltpu.async_copy(
                w16f_h.at[pl.ds(base * 16, CK * 16)], w16_v.at[b], semW[b])

        def step(b, g):
            pltpu.make_async_copy(
                tab_h.at[sidx_v.at[b]], row_v.at[b], semR[b]).wait()
            pltpu.make_async_copy(
                w16f_h.at[pl.ds(0, CK * 16)], w16_v.at[b], semW[b]).wait()

            def grp(l, c2):
                d16 = didx_v[b, pl.ds(16 * l, 16)]
                rec = plsc.load_gather(rec_v, [d16])
                wv = plsc.load_gather(w16_v.at[b], [256 * l + ii * 16 + 1])
                alpha = wv * rec
                for i in range(16):
                    a = alpha[i]
                    e = 16 * l + i
                    for j in range(NJ):
                        sl = pl.ds(16 * j, 16)
                        row_v[b, e, sl] = row_v[b, e, sl] * a
                return c2
            lax.fori_loop(0, CK // 16, grp, 0)
            pltpu.sync_copy(row_v.at[b], z_sh.at[didx_v.at[b]], add=True)

            @pl.when(g + 2 < NCH)
            def _():
                issue(b, g + 2)

        _run_chunks(NCH, issue, step)
        plsc.subcore_barrier()
        _dump_shared(z_sh, out_h, cid, sid, RPT, RZB)

    return k(tab, w16f, src_idx, dst_idx, recip)


def kernel(x, word2vec, s_f, edge_index, W_e, b_e, W_el, b_el,
           W_a, b_a, W_al, b_al, W_n, b_n, W_nl, b_nl):
    N, D = x.shape
    DW = word2vec.shape[1]
    E, DS = s_f.shape
    DWP = DW + 4
    LA = 112
    LB = 112
    LC = DWP - LA - LB
    NP = 10240

    src_idx = edge_index[0].astype(jnp.int32)
    dst_idx = edge_index[1].astype(jnp.int32)

    WeT = W_e[0:D]
    WeM = W_e[D:D + DS]
    WeB = W_e[D + DS:]
    WelT = jnp.pad(W_el[0:DW], ((0, 0), (0, DWP - DW)))
    WelB = jnp.pad(W_el[DW:], ((0, 0), (0, DWP - DW)))
    belp = jnp.pad(b_el, (0, DWP - DW)).reshape(1, DWP)
    be2 = b_e.reshape(1, D)
    wa = W_a[:, 0]
    wal = jnp.pad(W_al[:, 0], (0, DWP - DW))
    params = jnp.concatenate(
        [b_a.astype(F32), b_al.astype(F32), jnp.zeros((14,), F32)])
    w2vA = word2vec[:, 0:LA]
    w2vB = word2vec[:, LA:LA + LB]
    w2vC = jnp.pad(word2vec[:, LA + LB:], ((0, 0), (0, DWP - DW)))
    WnT = W_n[0:D]
    WnBA = W_n[D:D + D // 2]
    WnBB = W_n[D + D // 2:]
    bn2 = b_n.reshape(1, D)
    WnlT = W_nl[0:DW]
    WnlBA = W_nl[DW:DW + LA]
    WnlBB = W_nl[DW + LA:DW + LA + LB]
    WnlBC = jnp.pad(W_nl[DW + LA + LB:], ((0, DWP - DW), (0, 0)))
    bnl2 = b_nl.reshape(1, DW)

    ax_tab, b_tab, p_tab, q_tab = _t1_tables(
        x, word2vec, WeT, WeB, WelT, WelB, belp)
    cedge = _t2_cedge(s_f, WeM, be2)
    wA, zrawA, zrawB, s16a = _p1_feat(
        ax_tab, b_tab, cedge, src_idx, dst_idx, wa, params, NP)
    wB, s16b = _p1_lang(p_tab, q_tab, src_idx, dst_idx, wal, params, NP)
    recip = _t3_recip(s16a, s16b)
    wAf = wA.reshape(E * 16)
    wBf = wB.reshape(E * 16)
    zfA = _p2_zf(zrawA, wAf, dst_idx, recip, 64)
    zfB = _p2_zf(zrawB, wAf, dst_idx, recip, 64)
    za = _p3_lang(w2vA, wBf, src_idx, dst_idx, recip, LA)
    zb = _p3_lang(w2vB, wBf, src_idx, dst_idx, recip, LB)
    zc = _p3_lang(w2vC, wBf, src_idx, dst_idx, recip, LC)
    o1, o2 = _t4_node(
        x, word2vec, zfA, zfB, za, zb, zc,
        WnT, WnBA, WnBB, bn2, WnlT, WnlBA, WnlBB, WnlBC, bnl2)
    return (o1, o2)
